# baseline (speedup 1.0000x reference)
import jax
import jax.numpy as jnp
from jax.experimental import pallas as pl
from jax.experimental.pallas import tpu as pltpu

_B, _A, _F = 1024, 128, 75
_C = 16
_H = 33
_OUT = 64


def _body(n_ref, x_ref, wt_ref, o_ref):
    mb = x_ref.shape[1]
    x = x_ref[...]
    cls = x[0:_C]
    m = jnp.max(cls, axis=0)
    fidx = jax.lax.broadcasted_iota(jnp.int32, (_C, mb, _A), 0)
    cand = jnp.where(cls == m[None], fidx, _C)
    p = jnp.min(cand, axis=0)
    onehot = jnp.where(fidx == p[None], 1.0, 0.0)
    com = jnp.dot(wt_ref[...], onehot.reshape(_C, mb * _A),
                  preferred_element_type=jnp.float32,
                  precision=jax.lax.Precision.HIGHEST)
    oth = x[44:_F]
    ss = jnp.sum(oth * oth, axis=0)
    inv = jax.lax.rsqrt(jnp.maximum(ss, 1e-24))
    tf = oth * inv[None]
    y = jnp.concatenate([com.reshape(_H, mb, _A), tf], axis=0)
    mols = pl.program_id(0) * mb + jax.lax.broadcasted_iota(jnp.int32, (mb, _A), 0)
    y = jnp.where((mols < n_ref[0])[None], y, 0.0)
    o_ref[...] = jnp.transpose(y, (1, 0, 2))


def kernel(molecule_atoms, W, N):
    xt = jnp.transpose(molecule_atoms, (2, 0, 1))
    n_arr = jnp.asarray(N, jnp.int32).reshape(1)
    mb = 32
    grid = _B // mb
    out = pl.pallas_call(
        _body,
        grid=(grid,),
        in_specs=[
            pl.BlockSpec(memory_space=pltpu.SMEM),
            pl.BlockSpec((_F, mb, _A), lambda i: (0, i, 0)),
            pl.BlockSpec((_H, _C), lambda i: (0, 0)),
        ],
        out_specs=pl.BlockSpec((mb, _OUT, _A), lambda i: (i, 0, 0)),
        out_shape=jax.ShapeDtypeStruct((_B, _OUT, _A), jnp.float32),
    )(n_arr, xt, W.T)
    return jnp.transpose(out, (0, 2, 1))


# speedup vs baseline: 1.3775x; 1.3775x over previous
import jax
import jax.numpy as jnp
from jax.experimental import pallas as pl
from jax.experimental.pallas import tpu as pltpu

_B, _A, _F = 1024, 128, 75
_C = 16
_H = 33
_OUT = 64


def _body(n_ref, x_ref, wt_ref, o_ref):
    mb = x_ref.shape[1]
    x = x_ref[...]
    cls = x[0:_C]
    m = jnp.max(cls, axis=0)
    fidx = jax.lax.broadcasted_iota(jnp.int32, (_C, mb, _A), 0)
    cand = jnp.where(cls == m[None], fidx, _C)
    p = jnp.min(cand, axis=0)
    onehot = jnp.where(fidx == p[None], 1.0, 0.0)
    com = jnp.dot(wt_ref[...], onehot.reshape(_C, mb * _A),
                  preferred_element_type=jnp.float32,
                  precision=jax.lax.Precision.HIGHEST)
    oth = x[44:_F]
    ss = jnp.sum(oth * oth, axis=0)
    inv = jax.lax.rsqrt(jnp.maximum(ss, 1e-24))
    tf = oth * inv[None]
    y = jnp.concatenate([com.reshape(_H, mb, _A), tf], axis=0)
    mols = pl.program_id(0) * mb + jax.lax.broadcasted_iota(jnp.int32, (mb, _A), 0)
    y = jnp.where((mols < n_ref[0])[None], y, 0.0)
    o_ref[...] = jnp.transpose(y, (1, 0, 2))


def kernel(molecule_atoms, W, N):
    xt = jnp.transpose(molecule_atoms, (2, 0, 1))
    n_arr = jnp.asarray(N, jnp.int32).reshape(1)
    mb = 128
    grid = _B // mb
    out = pl.pallas_call(
        _body,
        grid=(grid,),
        in_specs=[
            pl.BlockSpec(memory_space=pltpu.SMEM),
            pl.BlockSpec((_F, mb, _A), lambda i: (0, i, 0)),
            pl.BlockSpec((_H, _C), lambda i: (0, 0)),
        ],
        out_specs=pl.BlockSpec((mb, _OUT, _A), lambda i: (i, 0, 0)),
        out_shape=jax.ShapeDtypeStruct((_B, _OUT, _A), jnp.float32),
    )(n_arr, xt, W.T)
    return jnp.transpose(out, (0, 2, 1))


# speedup vs baseline: 1.3856x; 1.0059x over previous
import jax
import jax.numpy as jnp
from jax.experimental import pallas as pl
from jax.experimental.pallas import tpu as pltpu

_B, _A, _F = 1024, 128, 75
_C = 16
_H = 33
_OUT = 64


def _body(n_ref, x_ref, wt_ref, o_ref):
    mb = x_ref.shape[1]
    x = x_ref[...]
    cls = x[0:_C]
    m = jnp.max(cls, axis=0)
    fidx = jax.lax.broadcasted_iota(jnp.int32, (_C, mb, _A), 0)
    cand = jnp.where(cls == m[None], fidx, _C)
    p = jnp.min(cand, axis=0)
    onehot = jnp.where(fidx == p[None], 1.0, 0.0)
    com = jnp.dot(wt_ref[...], onehot.reshape(_C, mb * _A),
                  preferred_element_type=jnp.float32,
                  precision=jax.lax.Precision.HIGHEST)
    oth = x[44:_F]
    ss = jnp.sum(oth * oth, axis=0)
    inv = jax.lax.rsqrt(jnp.maximum(ss, 1e-24))
    tf = oth * inv[None]
    y = jnp.concatenate([com.reshape(_H, mb, _A), tf], axis=0)
    mols = pl.program_id(0) * mb + jax.lax.broadcasted_iota(jnp.int32, (mb, _A), 0)
    y = jnp.where((mols < n_ref[0])[None], y, 0.0)
    o_ref[...] = jnp.transpose(y, (1, 0, 2))


def kernel(molecule_atoms, W, N):
    xt = jnp.transpose(molecule_atoms, (2, 0, 1))
    n_arr = jnp.asarray(N, jnp.int32).reshape(1)
    mb = 256
    grid = _B // mb
    out = pl.pallas_call(
        _body,
        grid=(grid,),
        in_specs=[
            pl.BlockSpec(memory_space=pltpu.SMEM),
            pl.BlockSpec((_F, mb, _A), lambda i: (0, i, 0)),
            pl.BlockSpec((_H, _C), lambda i: (0, 0)),
        ],
        out_specs=pl.BlockSpec((mb, _OUT, _A), lambda i: (i, 0, 0)),
        out_shape=jax.ShapeDtypeStruct((_B, _OUT, _A), jnp.float32),
    )(n_arr, xt, W.T)
    return jnp.transpose(out, (0, 2, 1))
